# baseline (speedup 1.0000x reference)
import jax
import jax.numpy as jnp
from jax.experimental import pallas as pl

N = 2048
_F32 = jnp.float32
_BF16 = jnp.bfloat16


def _dot(a, b):
    return jnp.dot(a, b, preferred_element_type=_F32)


def _relu(x):
    return jnp.maximum(x, 0.0)


def _recon_branch_kernel(adj_ref, p0_ref, dw2_ref, dw3_ref, recon_ref):
    a = adj_ref[...]

    def hop(x16, w_ref):
        p = _dot(x16, w_ref[...].astype(_BF16))
        return _dot(a, p.astype(_BF16))

    r = _relu(_dot(a, p0_ref[...])).astype(_BF16)
    r = _relu(hop(r, dw2_ref)).astype(_BF16)
    recon_ref[...] = hop(r, dw3_ref)


def _recon_branch(adj16, p0, dw2, dw3):
    return pl.pallas_call(
        _recon_branch_kernel,
        out_shape=jax.ShapeDtypeStruct((N, dw3.shape[1]), _F32))(
            adj16, p0, dw2, dw3)


def _recon_branch_spa_kernel(adj_ref, p0_ref, ps_ref, dw2_ref, dw3_ref,
                             recon_ref, rspa_ref):
    a = adj_ref[...]

    def hop(x16, w_ref):
        p = _dot(x16, w_ref[...].astype(_BF16))
        return _dot(a, p.astype(_BF16))

    rspa_ref[...] = _dot(a, ps_ref[...])
    r = _relu(_dot(a, p0_ref[...])).astype(_BF16)
    r = _relu(hop(r, dw2_ref)).astype(_BF16)
    recon_ref[...] = hop(r, dw3_ref)


def _recon_branch_spa(adj16, p0, ps, dw2, dw3):
    out_shape = [
        jax.ShapeDtypeStruct((N, dw3.shape[1]), _F32),
        jax.ShapeDtypeStruct((N, ps.shape[1]), _F32),
    ]
    return pl.pallas_call(_recon_branch_spa_kernel, out_shape=out_shape)(
        adj16, p0, ps, dw2, dw3)


def _cross_branch_kernel(adj_ref, p0_ref, dw2_ref, dw3_ref,
                         ew1_ref, ew2_ref, ew3_ref, cross_ref):
    a = adj_ref[...]

    def hop(x16, w_ref):
        p = _dot(x16, w_ref[...].astype(_BF16))
        return _dot(a, p.astype(_BF16))

    c = _relu(_dot(a, p0_ref[...])).astype(_BF16)
    c = _relu(hop(c, dw2_ref)).astype(_BF16)
    c = hop(c, dw3_ref).astype(_BF16)
    c = _relu(hop(c, ew1_ref)).astype(_BF16)
    c = _relu(hop(c, ew2_ref)).astype(_BF16)
    cross_ref[...] = hop(c, ew3_ref)


def _cross_branch(adj16, p0, dw2, dw3, ew1, ew2, ew3):
    return pl.pallas_call(
        _cross_branch_kernel,
        out_shape=jax.ShapeDtypeStruct((N, ew3.shape[1]), _F32))(
            adj16, p0, dw2, dw3, ew1, ew2, ew3)


def _recon_kernel(adj_ref, p_ref, o_ref):
    o_ref[...] = _dot(adj_ref[...], p_ref[...])


def _recon_hop(adj, p):
    return pl.pallas_call(
        _recon_kernel,
        out_shape=jax.ShapeDtypeStruct((N, p.shape[1]), _F32))(adj, p)


def kernel(f_omics1, f_omics2, adj_spa1, adj_fea1, adj_spa2, adj_fea2,
           cell_emb, adj_emb, W_emb_enc, W_emb_dec,
           enc1_W1, enc1_W2, enc1_W3, dec1_W1, dec1_W2, dec1_W3,
           enc2_W1, enc2_W2, enc2_W3, dec2_W1, dec2_W2, dec2_W3,
           att1_w, att1_u, att2_w, att2_u, attf_w, attf_u,
           atto2_w, atto2_u, attc_w, attc_u):
    def _deep(feat, adj, W1, W2, W3):
        x = jax.nn.relu(adj @ (feat @ W1))
        x = jax.nn.relu(adj @ (x @ W2))
        return adj @ (x @ W3)

    def _att(e1, e2, w, u):
        s = jnp.stack([e1, e2], axis=1)
        v = jnp.tanh(s @ w)
        vu = jnp.squeeze(v @ u, -1)
        alpha = jax.nn.softmax(vu + 1e-06, axis=1)
        comb = jnp.squeeze(
            jnp.matmul(jnp.transpose(s, (0, 2, 1)), alpha[:, :, None]), -1)
        return comb, alpha

    c = cell_emb @ W_emb_enc
    emb_spa = adj_spa1 @ c
    emb_fea = adj_emb @ c
    latent_spa1 = _deep(f_omics1, adj_spa1, enc1_W1, enc1_W2, enc1_W3)
    latent_spa2 = _deep(f_omics2, adj_spa2, enc2_W1, enc2_W2, enc2_W3)
    latent_fea1 = _deep(f_omics1, adj_fea1, enc1_W1, enc1_W2, enc1_W3)
    latent_fea2 = _deep(f_omics2, adj_fea2, enc2_W1, enc2_W2, enc2_W3)
    emb_att1, alpha_att1 = _att(emb_spa, latent_spa1, att1_w, att1_u)
    emb_att2, alpha_att2 = _att(emb_fea, latent_fea1, att2_w, att2_u)
    o1, alpha_att_omics1 = _att(emb_att1, emb_att2, attf_w, attf_u)
    o2, alpha_omics2 = _att(latent_spa2, latent_fea2, atto2_w, atto2_u)
    comb, alpha = _att(o1, o2, attc_w, attc_u)

    p_r1 = (comb @ dec1_W1).astype(_BF16)
    p_c2 = (o2 @ dec1_W1).astype(_BF16)
    p_r2 = (comb @ dec2_W1).astype(_BF16)
    p_c1 = (o1 @ dec2_W1).astype(_BF16)
    p_rs = (emb_spa @ W_emb_dec).astype(_BF16)
    p_rf = emb_fea @ W_emb_dec

    spa1_16 = adj_spa1.astype(_BF16)
    spa2_16 = adj_spa2.astype(_BF16)

    emb_recon1, recon_spa = _recon_branch_spa(
        spa1_16, p_r1, p_rs, dec1_W2, dec1_W3)
    emb_cross2 = _cross_branch(
        spa1_16, p_c2, dec1_W2, dec1_W3, enc1_W1, enc1_W2, enc1_W3)
    emb_recon2 = _recon_branch(spa2_16, p_r2, dec2_W2, dec2_W3)
    emb_cross1 = _cross_branch(
        spa2_16, p_c1, dec2_W2, dec2_W3, enc2_W1, enc2_W2, enc2_W3)
    recon_fea = _recon_hop(adj_emb, p_rf)

    return (o1, o2, comb, emb_recon1, emb_recon2, emb_cross1, emb_cross2,
            alpha_att1, alpha_att2, alpha_att_omics1, alpha_omics2, alpha,
            recon_spa, recon_fea)
